# Initial kernel scaffold; baseline (speedup 1.0000x reference)
import jax
import jax.numpy as jnp
from jax.experimental import pallas as pl


def kernel(feature, sen_edge, resistant_edge, nb_celllines, nb_drugs, W_hg1, b_hg1, W_hg2, b_hg2, pipe_emb, W_proj, b_proj, W_cell, b_cell, W_drug, b_drug):
    raise NotImplementedError("write your pallas kernel here")


# speedup vs baseline: 5.2285x; 5.2285x over previous
import functools

import jax
import jax.numpy as jnp
from jax import lax
from jax.experimental import pallas as pl
from jax.experimental.pallas import tpu as pltpu
from jax.experimental.pallas import tpu_sc as plsc

N = 10000
E = 320000
D = 128
TAU = 10.0
THRESHOLD = 0.8
NEG_SLOPE = 0.2

_NC = 2
_NS = 16
_CH = 128
_EPT = E // _NS
_NCHUNK = -(-_EPT // _CH)
_EPAD = _NCHUNK * _CH - _EPT
_RPT = 624
_RLAST = N - 15 * _RPT
_DPT = 640
_DLAST = N - 15 * _DPT


def _seg_sum_body(with_deg, *refs):
    if with_deg:
        (x_hbm, src_hbm, dst_hbm, z2_hbm,
         out_hbm, deg_hbm,
         acc_sh, deg_sh, src_v, dstbuf_v, rows_v, ones_v, degbuf_v,
         sem) = refs
    else:
        (x_hbm, src_hbm, dst_hbm, z2_hbm,
         out_hbm,
         acc_sh, src_v, dstbuf_v, rows_v, sem) = refs
    c = lax.axis_index("c")
    s = lax.axis_index("s")
    @pl.when(s < _NS - 1)
    def _():
        pltpu.sync_copy(z2_hbm.at[pl.ds(s * _RPT, _RPT)],
                        acc_sh.at[pl.ds(s * _RPT, _RPT)])

    @pl.when(s == _NS - 1)
    def _():
        pltpu.sync_copy(z2_hbm.at[pl.ds(15 * _RPT, _RLAST)],
                        acc_sh.at[pl.ds(15 * _RPT, _RLAST)])
    if with_deg:
        def zero16(j, carry):
            degbuf_v[pl.ds(j * 16, 16)] = jnp.zeros((16,), jnp.float32)
            return carry
        lax.fori_loop(0, _DPT // 16, zero16, 0)

        @pl.when(s < _NS - 1)
        def _():
            pltpu.sync_copy(degbuf_v, deg_sh.at[pl.ds(s * _DPT, _DPT)])

        @pl.when(s == _NS - 1)
        def _():
            pltpu.sync_copy(degbuf_v.at[pl.ds(0, _DLAST)],
                            deg_sh.at[pl.ds(15 * _DPT, _DLAST)])
        for j in range(_CH // 16):
            ones_v[pl.ds(j * 16, 16)] = jnp.full((16,), 1.0, jnp.float32)
    pltpu.sync_copy(src_hbm.at[c, s], src_v)
    plsc.subcore_barrier()

    def chunk(j, carry):
        pltpu.sync_copy(dst_hbm.at[c, s, j], dstbuf_v)
        pltpu.async_copy(x_hbm.at[src_v.at[j]], rows_v, sem).wait()
        pltpu.sync_copy(rows_v, acc_sh.at[dstbuf_v], add=True)
        if with_deg:
            pltpu.sync_copy(ones_v, deg_sh.at[dstbuf_v], add=True)
        return carry

    lax.fori_loop(0, _NCHUNK, chunk, 0)
    plsc.subcore_barrier()

    @pl.when(s < _NS - 1)
    def _():
        pltpu.sync_copy(acc_sh.at[pl.ds(s * _RPT, _RPT)],
                        out_hbm.at[c, pl.ds(s * _RPT, _RPT)])

    @pl.when(s == _NS - 1)
    def _():
        pltpu.sync_copy(acc_sh.at[pl.ds(15 * _RPT, _RLAST)],
                        out_hbm.at[c, pl.ds(15 * _RPT, _RLAST)])
    if with_deg:
        @pl.when(s < _NS - 1)
        def _():
            pltpu.sync_copy(deg_sh.at[pl.ds(s * _DPT, _DPT)], degbuf_v)
            pltpu.sync_copy(degbuf_v,
                            deg_hbm.at[pl.ds(c * N + s * _DPT, _DPT)])

        @pl.when(s == _NS - 1)
        def _():
            pltpu.sync_copy(deg_sh.at[pl.ds(15 * _DPT, _DLAST)],
                            degbuf_v.at[pl.ds(0, _DLAST)])
            pltpu.sync_copy(degbuf_v.at[pl.ds(0, _DLAST)],
                            deg_hbm.at[pl.ds(c * N + 15 * _DPT, _DLAST)])


@functools.lru_cache(maxsize=None)
def _make_seg(with_deg):
    mesh = plsc.VectorSubcoreMesh(core_axis_name="c", subcore_axis_name="s")
    if with_deg:
        out_type = (jax.ShapeDtypeStruct((_NC, N, D), jnp.float32),
                    jax.ShapeDtypeStruct((_NC * N,), jnp.float32))
        scratch = [
            pltpu.VMEM_SHARED((N + 8, D), jnp.float32),
            pltpu.VMEM_SHARED((N + 16,), jnp.float32),
            pltpu.VMEM((_NCHUNK, _CH), jnp.int32),
            pltpu.VMEM((_CH,), jnp.int32),
            pltpu.VMEM((_CH, D), jnp.float32),
            pltpu.VMEM((_CH,), jnp.float32),
            pltpu.VMEM((_DPT,), jnp.float32),
            pltpu.SemaphoreType.DMA,
        ]
    else:
        out_type = jax.ShapeDtypeStruct((_NC, N, D), jnp.float32)
        scratch = [
            pltpu.VMEM_SHARED((N + 8, D), jnp.float32),
            pltpu.VMEM((_NCHUNK, _CH), jnp.int32),
            pltpu.VMEM((_CH,), jnp.int32),
            pltpu.VMEM((_CH, D), jnp.float32),
            pltpu.SemaphoreType.DMA,
        ]
    return functools.partial(
        pl.kernel, mesh=mesh, out_type=out_type, scratch_types=scratch,
    )(functools.partial(_seg_sum_body, with_deg))


def _dot(a, b, dims=(((1,), (0,)), ((), ()))):
    return lax.dot_general(a, b, dims,
                           precision=lax.Precision.HIGHEST,
                           preferred_element_type=jnp.float32)


def _layer_tc_body(acc_ref, deg_ref, w_ref, b_ref, o_ref):
    inv = 1.0 / jnp.maximum(deg_ref[...], 1.0)
    m = acc_ref[0] * inv[0] + acc_ref[1] * inv[1]
    h = _dot(m, w_ref[...]) + b_ref[...]
    o_ref[...] = jnp.where(h >= 0.0, h, NEG_SLOPE * h)


_layer_tc = pl.pallas_call(
    _layer_tc_body,
    out_shape=jax.ShapeDtypeStruct((N, D), jnp.float32))


def _final_tc_body(x_ref, pipe_ref,
                   wp_ref, bp_ref, wc_ref, bc_ref, wd_ref, bd_ref,
                   mapped_ref, xout_ref):
    x = x_ref[...]
    logits = _dot(x, pipe_ref[...], (((1,), (1,)), ((), ()))) * (1.0 / TAU)
    mx = jnp.max(logits, axis=-1, keepdims=True)
    ex = jnp.exp(logits - mx)
    p = ex / jnp.sum(ex, axis=-1, keepdims=True)
    pm = jnp.where(p >= THRESHOLD * jnp.max(p, axis=-1, keepdims=True),
                   p, 0.0)
    colsum = jnp.sum(pm, axis=0)
    pmtx = _dot(pm, x, (((0,), (0,)), ((), ())))
    pm2 = pm * (1.0 / (colsum + 1e-8))[None, :]
    x = x + _dot(pm2, pmtx)
    mapped = _dot(x, wp_ref[...]) + bp_ref[...]
    mapped_ref[...] = mapped
    xc = _dot(mapped[0:8000], wc_ref[...]) + bc_ref[...]
    xd = _dot(mapped[8000:N], wd_ref[...]) + bd_ref[...]
    xout_ref[0:8000, :] = jnp.maximum(xc, 0.0)
    xout_ref[8000:N, :] = jnp.maximum(xd, 0.0)


_final_tc = pl.pallas_call(
    _final_tc_body,
    out_shape=(jax.ShapeDtypeStruct((N, D), jnp.float32),
               jax.ShapeDtypeStruct((N, D), jnp.float32)))


def kernel(feature, sen_edge, resistant_edge, nb_celllines, nb_drugs,
           W_hg1, b_hg1, W_hg2, b_hg2, pipe_emb,
           W_proj, b_proj, W_cell, b_cell, W_drug, b_drug):
    del nb_celllines, nb_drugs
    src = jnp.pad(
        jnp.stack([sen_edge[0], resistant_edge[0]]).reshape(_NC, _NS, _EPT),
        ((0, 0), (0, 0), (0, _EPAD)), constant_values=0,
    ).reshape(_NC, _NS, _NCHUNK, _CH).astype(jnp.int32)
    dst = jnp.pad(
        jnp.stack([sen_edge[1], resistant_edge[1]]).reshape(_NC, _NS, _EPT),
        ((0, 0), (0, 0), (0, _EPAD)), constant_values=N,
    ).reshape(_NC, _NS, _NCHUNK, _CH).astype(jnp.int32)
    z2 = jnp.zeros((N, D), jnp.float32)

    msum1, deg = _make_seg(True)(feature, src, dst, z2)
    deg3 = deg.reshape(_NC, N, 1)
    b1 = jnp.reshape(b_hg1, (1, D))
    x1 = _layer_tc(msum1, deg3, W_hg1, b1)
    msum2 = _make_seg(False)(x1, src, dst, z2)
    x2 = _layer_tc(msum2, deg3, W_hg2, jnp.reshape(b_hg2, (1, D)))
    mapped, xout = _final_tc(
        x2, pipe_emb,
        W_proj, jnp.reshape(b_proj, (1, D)),
        W_cell, jnp.reshape(b_cell, (1, D)),
        W_drug, jnp.reshape(b_drug, (1, D)))
    return (mapped, xout)


# speedup vs baseline: 8.8562x; 1.6938x over previous
import functools

import jax
import jax.numpy as jnp
from jax import lax
from jax.experimental import pallas as pl
from jax.experimental.pallas import tpu as pltpu
from jax.experimental.pallas import tpu_sc as plsc

N = 10000
E = 320000
D = 128
TAU = 10.0
THRESHOLD = 0.8
NEG_SLOPE = 0.2

_NC = 2
_NS = 16
_CH = 128
_EPT = E // _NS
_NFULL = _EPT // _CH
_NPAIR = _NFULL // 2
_TAIL = _EPT - _NFULL * _CH
_RPT = 624
_RLAST = N - 15 * _RPT
_DPT = 640
_DLAST = N - 15 * _DPT


def _seg_sum_body(with_deg, *refs):
    if with_deg:
        (x_hbm, sen_hbm, res_hbm,
         out_hbm, deg_hbm,
         acc_sh, deg_sh, rows0, rows1, srcb0, srcb1, dstb0, dstb1,
         dsttail, ones_v, degbuf_v, gsem0, gsem1) = refs
    else:
        (x_hbm, sen_hbm, res_hbm,
         out_hbm,
         acc_sh, rows0, rows1, srcb0, srcb1, dstb0, dstb1,
         dsttail, gsem0, gsem1) = refs
    c = lax.axis_index("c")
    s = lax.axis_index("s")
    rows = (rows0, rows1)
    srcb = (srcb0, srcb1)
    dstb = (dstb0, dstb1)
    gsem = (gsem0, gsem1)

    def zrow(i, carry):
        for jj in range(D // 16):
            rows0[i, pl.ds(jj * 16, 16)] = jnp.zeros((16,), jnp.float32)
        return carry
    lax.fori_loop(0, _CH, zrow, 0)

    @pl.when(s < _NS - 1)
    def _():
        for q in range(4):
            pltpu.sync_copy(rows0, acc_sh.at[pl.ds(s * _RPT + q * _CH, _CH)])
        pltpu.sync_copy(rows0.at[pl.ds(0, _RPT - 4 * _CH)],
                        acc_sh.at[pl.ds(s * _RPT + 4 * _CH, _RPT - 4 * _CH)])

    @pl.when(s == _NS - 1)
    def _():
        for q in range(5):
            pltpu.sync_copy(rows0, acc_sh.at[pl.ds(15 * _RPT + q * _CH, _CH)])

    if with_deg:
        def zdeg(j, carry):
            degbuf_v[pl.ds(j * 16, 16)] = jnp.zeros((16,), jnp.float32)
            return carry
        lax.fori_loop(0, _DPT // 16, zdeg, 0)

        @pl.when(s < _NS - 1)
        def _():
            pltpu.sync_copy(degbuf_v, deg_sh.at[pl.ds(s * _DPT, _DPT)])

        @pl.when(s == _NS - 1)
        def _():
            pltpu.sync_copy(degbuf_v.at[pl.ds(0, _DLAST)],
                            deg_sh.at[pl.ds(15 * _DPT, _DLAST)])
        for j in range(_CH // 16):
            ones_v[pl.ds(j * 16, 16)] = jnp.full((16,), 1.0, jnp.float32)

    plsc.subcore_barrier()

    def run_relation(edge_ref):
        base = s * _EPT
        for b in range(2):
            pltpu.sync_copy(edge_ref.at[pl.ds(base + b * _CH, _CH)], srcb[b])
            pltpu.sync_copy(edge_ref.at[pl.ds(E + base + b * _CH, _CH)],
                            dstb[b])
            pltpu.async_copy(x_hbm.at[srcb[b]], rows[b], gsem[b])

        def pair(j, carry):
            for b in range(2):
                pltpu.make_async_copy(x_hbm.at[srcb[b]], rows[b],
                                      gsem[b]).wait()
                pltpu.sync_copy(rows[b], acc_sh.at[dstb[b]], add=True)
                if with_deg:
                    pltpu.sync_copy(ones_v, deg_sh.at[dstb[b]], add=True)

                @pl.when(j < _NPAIR - 1)
                def _():
                    off = base + (2 * j + b + 2) * _CH
                    pltpu.sync_copy(edge_ref.at[pl.ds(off, _CH)], srcb[b])
                    pltpu.sync_copy(edge_ref.at[pl.ds(E + off, _CH)], dstb[b])
                    pltpu.async_copy(x_hbm.at[srcb[b]], rows[b], gsem[b])
            return carry

        lax.fori_loop(0, _NPAIR, pair, 0)
        toff = base + _NFULL * _CH
        pltpu.sync_copy(edge_ref.at[pl.ds(toff, _TAIL)],
                        srcb[0].at[pl.ds(0, _TAIL)])
        pltpu.sync_copy(edge_ref.at[pl.ds(E + toff, _TAIL)], dsttail)
        pltpu.async_copy(x_hbm.at[srcb[0].at[pl.ds(0, _TAIL)]],
                         rows0.at[pl.ds(0, _TAIL)], gsem0).wait()
        pltpu.sync_copy(rows0.at[pl.ds(0, _TAIL)], acc_sh.at[dsttail],
                        add=True)
        if with_deg:
            pltpu.sync_copy(ones_v.at[pl.ds(0, _TAIL)], deg_sh.at[dsttail],
                            add=True)

    @pl.when(c == 0)
    def _():
        run_relation(sen_hbm)

    @pl.when(c == 1)
    def _():
        run_relation(res_hbm)

    plsc.subcore_barrier()

    @pl.when(s < _NS - 1)
    def _():
        pltpu.sync_copy(acc_sh.at[pl.ds(s * _RPT, _RPT)],
                        out_hbm.at[c, pl.ds(s * _RPT, _RPT)])

    @pl.when(s == _NS - 1)
    def _():
        pltpu.sync_copy(acc_sh.at[pl.ds(15 * _RPT, _RLAST)],
                        out_hbm.at[c, pl.ds(15 * _RPT, _RLAST)])
    if with_deg:
        @pl.when(s < _NS - 1)
        def _():
            pltpu.sync_copy(deg_sh.at[pl.ds(s * _DPT, _DPT)], degbuf_v)
            pltpu.sync_copy(degbuf_v,
                            deg_hbm.at[pl.ds(c * N + s * _DPT, _DPT)])

        @pl.when(s == _NS - 1)
        def _():
            pltpu.sync_copy(deg_sh.at[pl.ds(15 * _DPT, _DLAST)],
                            degbuf_v.at[pl.ds(0, _DLAST)])
            pltpu.sync_copy(degbuf_v.at[pl.ds(0, _DLAST)],
                            deg_hbm.at[pl.ds(c * N + 15 * _DPT, _DLAST)])


@functools.lru_cache(maxsize=None)
def _make_seg(with_deg):
    mesh = plsc.VectorSubcoreMesh(core_axis_name="c", subcore_axis_name="s")
    if with_deg:
        out_type = (jax.ShapeDtypeStruct((_NC, N, D), jnp.float32),
                    jax.ShapeDtypeStruct((_NC * N,), jnp.float32))
        scratch = [
            pltpu.VMEM_SHARED((N + 8, D), jnp.float32),
            pltpu.VMEM_SHARED((N + 16,), jnp.float32),
            pltpu.VMEM((_CH, D), jnp.float32),
            pltpu.VMEM((_CH, D), jnp.float32),
            pltpu.VMEM((_CH,), jnp.int32),
            pltpu.VMEM((_CH,), jnp.int32),
            pltpu.VMEM((_CH,), jnp.int32),
            pltpu.VMEM((_CH,), jnp.int32),
            pltpu.VMEM((_TAIL,), jnp.int32),
            pltpu.VMEM((_CH,), jnp.float32),
            pltpu.VMEM((_DPT,), jnp.float32),
            pltpu.SemaphoreType.DMA,
            pltpu.SemaphoreType.DMA,
        ]
    else:
        out_type = jax.ShapeDtypeStruct((_NC, N, D), jnp.float32)
        scratch = [
            pltpu.VMEM_SHARED((N + 8, D), jnp.float32),
            pltpu.VMEM((_CH, D), jnp.float32),
            pltpu.VMEM((_CH, D), jnp.float32),
            pltpu.VMEM((_CH,), jnp.int32),
            pltpu.VMEM((_CH,), jnp.int32),
            pltpu.VMEM((_CH,), jnp.int32),
            pltpu.VMEM((_CH,), jnp.int32),
            pltpu.VMEM((_TAIL,), jnp.int32),
            pltpu.SemaphoreType.DMA,
            pltpu.SemaphoreType.DMA,
        ]
    return functools.partial(
        pl.kernel, mesh=mesh, out_type=out_type, scratch_types=scratch,
    )(functools.partial(_seg_sum_body, with_deg))


def _dot(a, b, dims=(((1,), (0,)), ((), ()))):
    return lax.dot_general(a, b, dims,
                           precision=lax.Precision.HIGHEST,
                           preferred_element_type=jnp.float32)


def _layer_tc_body(acc_ref, deg_ref, w_ref, b_ref, o_ref):
    inv = 1.0 / jnp.maximum(deg_ref[...], 1.0)
    m = acc_ref[0] * inv[0] + acc_ref[1] * inv[1]
    h = _dot(m, w_ref[...]) + b_ref[...]
    o_ref[...] = jnp.where(h >= 0.0, h, NEG_SLOPE * h)


_layer_tc = pl.pallas_call(
    _layer_tc_body,
    out_shape=jax.ShapeDtypeStruct((N, D), jnp.float32))


def _final_tc_body(x_ref, pipe_ref,
                   wp_ref, bp_ref, wc_ref, bc_ref, wd_ref, bd_ref,
                   mapped_ref, xout_ref):
    x = x_ref[...]
    logits = _dot(x, pipe_ref[...], (((1,), (1,)), ((), ()))) * (1.0 / TAU)
    mx = jnp.max(logits, axis=-1, keepdims=True)
    ex = jnp.exp(logits - mx)
    p = ex / jnp.sum(ex, axis=-1, keepdims=True)
    pm = jnp.where(p >= THRESHOLD * jnp.max(p, axis=-1, keepdims=True),
                   p, 0.0)
    colsum = jnp.sum(pm, axis=0)
    pmtx = _dot(pm, x, (((0,), (0,)), ((), ())))
    pm2 = pm * (1.0 / (colsum + 1e-8))[None, :]
    x = x + _dot(pm2, pmtx)
    mapped = _dot(x, wp_ref[...]) + bp_ref[...]
    mapped_ref[...] = mapped
    xc = _dot(mapped[0:8000], wc_ref[...]) + bc_ref[...]
    xd = _dot(mapped[8000:N], wd_ref[...]) + bd_ref[...]
    xout_ref[0:8000, :] = jnp.maximum(xc, 0.0)
    xout_ref[8000:N, :] = jnp.maximum(xd, 0.0)


_final_tc = pl.pallas_call(
    _final_tc_body,
    out_shape=(jax.ShapeDtypeStruct((N, D), jnp.float32),
               jax.ShapeDtypeStruct((N, D), jnp.float32)))


def kernel(feature, sen_edge, resistant_edge, nb_celllines, nb_drugs,
           W_hg1, b_hg1, W_hg2, b_hg2, pipe_emb,
           W_proj, b_proj, W_cell, b_cell, W_drug, b_drug):
    del nb_celllines, nb_drugs
    sen_flat = sen_edge.reshape(-1).astype(jnp.int32)
    res_flat = resistant_edge.reshape(-1).astype(jnp.int32)

    msum1, deg = _make_seg(True)(feature, sen_flat, res_flat)
    deg3 = deg.reshape(_NC, N, 1)
    x1 = _layer_tc(msum1, deg3, W_hg1, jnp.reshape(b_hg1, (1, D)))
    msum2 = _make_seg(False)(x1, sen_flat, res_flat)
    x2 = _layer_tc(msum2, deg3, W_hg2, jnp.reshape(b_hg2, (1, D)))
    mapped, xout = _final_tc(
        x2, pipe_emb,
        W_proj, jnp.reshape(b_proj, (1, D)),
        W_cell, jnp.reshape(b_cell, (1, D)),
        W_drug, jnp.reshape(b_drug, (1, D)))
    return (mapped, xout)
